# baseline (speedup 1.0000x reference)
import functools

import jax
import jax.numpy as jnp
from jax import lax
from jax.experimental import pallas as pl
from jax.experimental.pallas import tpu as pltpu
from jax.experimental.pallas import tpu_sc as plsc

B = 16384
L = 200
OUT = 35

NW = 32
COLS_PER_W = B // NW
STAGE_ROWS = 40
NSTAGE = L // STAGE_ROWS
NPAIR = COLS_PER_W // 32

_ILV = plsc.PackFormat.INTERLEAVED


def _sc_body(x1_hbm, x2_hbm, consts_hbm, out_hbm,
             x1h0, x1h1, x2h0, x2h1, out_v, consts_v, mom_v,
             s1a, s1b, s2a, s2b, so):
    wid = lax.axis_index("s") * 2 + lax.axis_index("c")
    col0 = wid * COLS_PER_W

    pltpu.make_async_copy(consts_hbm, consts_v, so).start()
    pltpu.make_async_copy(consts_hbm, consts_v, so).wait()

    x_bufs = ([x1h0, x1h1], [x2h0, x2h1])
    sems = ([s1a, s1b], [s2a, s2b])

    def start(s):
        b = s % 2
        rows = s * STAGE_ROWS
        cp1 = pltpu.make_async_copy(
            x1_hbm.at[pl.ds(rows, STAGE_ROWS), pl.ds(col0, COLS_PER_W)],
            x_bufs[0][b], sems[0][b])
        cp2 = pltpu.make_async_copy(
            x2_hbm.at[pl.ds(rows, STAGE_ROWS), pl.ds(col0, COLS_PER_W)],
            x_bufs[1][b], sems[1][b])
        cp1.start()
        cp2.start()
        return cp1, cp2

    def accum(ref, offA, offB):
        pp = plsc.pack(ref[0, pl.ds(offA, 16)], ref[0, pl.ds(offB, 16)],
                       format=_ILV)
        s16 = pp
        q16 = pp * pp
        for r in range(1, STAGE_ROWS):
            pp = plsc.pack(ref[r, pl.ds(offA, 16)], ref[r, pl.ds(offB, 16)],
                           format=_ILV)
            s16 = s16 + pp
            q16 = q16 + pp * pp
        return s16, q16

    def mom_slot(k, p):
        return mom_v[k, pl.ds(p * 16, 16)]

    pending = {0: start(0)}
    for s in range(NSTAGE):
        if s + 1 < NSTAGE:
            pending[s + 1] = start(s + 1)
        cp1, cp2 = pending.pop(s)
        cp1.wait()
        cp2.wait()
        xb1, xb2 = x_bufs[0][s % 2], x_bufs[1][s % 2]

        if s == 0:
            def body0(p, carry):
                offA = p * 32
                s1, q1 = accum(xb1, offA, offA + 16)
                s2, q2 = accum(xb2, offA, offA + 16)
                for k, v in enumerate((s1, q1, s2, q2)):
                    mom_v[k, pl.ds(p * 16, 16)] = plsc.bitcast(v, jnp.int32)
                return carry
            lax.fori_loop(0, NPAIR, body0, 0)
        else:
            def body_mid(p, carry):
                offA = p * 32
                s1, q1 = accum(xb1, offA, offA + 16)
                s2, q2 = accum(xb2, offA, offA + 16)
                for k, v in enumerate((s1, q1, s2, q2)):
                    acc = plsc.bitcast(mom_slot(k, p), jnp.int16)
                    mom_v[k, pl.ds(p * 16, 16)] = plsc.bitcast(v + acc, jnp.int32)
                return carry
            lax.fori_loop(0, NPAIR, body_mid, 0)

    def body_emit(p, carry):
        offA = p * 32
        offB = offA + 16
        fA, fB = [], []
        for k in range(4):
            va, vb = plsc.unpack(plsc.bitcast(mom_slot(k, p), jnp.int16),
                                 format=_ILV)
            fA.append(va.astype(jnp.float32))
            fB.append(vb.astype(jnp.float32))
        def load_cb(j):
            return [consts_v[pl.ds((k * OUT + j) * 16, 16)] for k in range(5)]

        cb = load_cb(0)
        for j in range(OUT):
            nxt = load_cb(j + 1) if j + 1 < OUT else None
            out_v[j, pl.ds(offA, 16)] = (
                (cb[0] + fA[0] * cb[1]) + (fA[1] * cb[2] + fA[2] * cb[3])
                + fA[3] * cb[4])
            out_v[j, pl.ds(offB, 16)] = (
                (cb[0] + fB[0] * cb[1]) + (fB[1] * cb[2] + fB[2] * cb[3])
                + fB[3] * cb[4])
            cb = nxt
        return carry

    half_cols = COLS_PER_W // 2
    lax.fori_loop(0, NPAIR // 2, body_emit, 0)
    ocp1 = pltpu.make_async_copy(
        out_v.at[:, pl.ds(0, half_cols)],
        out_hbm.at[:, pl.ds(col0, half_cols)], so)
    ocp1.start()
    lax.fori_loop(NPAIR // 2, NPAIR, body_emit, 0)
    ocp2 = pltpu.make_async_copy(
        out_v.at[:, pl.ds(half_cols, half_cols)],
        out_hbm.at[:, pl.ds(col0 + half_cols, half_cols)], so)
    ocp2.start()
    ocp1.wait()
    ocp2.wait()


@jax.jit
def _run(x1t, x2t, consts):
    mesh = plsc.VectorSubcoreMesh(core_axis_name="c", subcore_axis_name="s")
    f = functools.partial(
        pl.kernel,
        mesh=mesh,
        compiler_params=pltpu.CompilerParams(needs_layout_passes=False),
        out_type=jax.ShapeDtypeStruct((OUT, B), jnp.float32),
        scratch_types=[
            pltpu.VMEM((STAGE_ROWS, COLS_PER_W), jnp.int32),
            pltpu.VMEM((STAGE_ROWS, COLS_PER_W), jnp.int32),
            pltpu.VMEM((STAGE_ROWS, COLS_PER_W), jnp.int32),
            pltpu.VMEM((STAGE_ROWS, COLS_PER_W), jnp.int32),
            pltpu.VMEM((OUT, COLS_PER_W), jnp.float32),
            pltpu.VMEM((5 * OUT * 16,), jnp.float32),
            pltpu.VMEM((4, COLS_PER_W // 2), jnp.int32),
            pltpu.SemaphoreType.DMA,
            pltpu.SemaphoreType.DMA,
            pltpu.SemaphoreType.DMA,
            pltpu.SemaphoreType.DMA,
            pltpu.SemaphoreType.DMA,
        ],
    )(_sc_body)
    return f(x1t, x2t, consts)


def kernel(x1_ids, x2_ids, emb1, emb2, W):
    e1 = emb1.astype(jnp.float32)
    e2 = emb2.astype(jnp.float32)
    z = jnp.zeros((4,), jnp.float32)
    lin1 = (-1.5 * e1[0] + 2.0 * e1[1] - 0.5 * e1[2]) / L
    quad1 = (0.5 * e1[0] - 1.0 * e1[1] + 0.5 * e1[2]) / L
    lin2 = (-1.5 * e2[0] + 2.0 * e2[1] - 0.5 * e2[2]) / L
    quad2 = (0.5 * e2[0] - 1.0 * e2[1] + 0.5 * e2[2]) / L
    G = jnp.stack([
        jnp.concatenate([e1[0], e2[0]]),
        jnp.concatenate([lin1, z]),
        jnp.concatenate([quad1, z]),
        jnp.concatenate([z, lin2]),
        jnp.concatenate([z, quad2]),
    ])
    consts = jax.lax.dot(G, W.astype(jnp.float32),
                         precision=jax.lax.Precision.HIGHEST)
    consts_b = jnp.broadcast_to(consts[:, :, None], (5, OUT, 16)).reshape(-1)

    out_t = _run(x1_ids.T, x2_ids.T, consts_b)
    return out_t.T
